# baseline (speedup 1.0000x reference)
import functools

import jax
import jax.numpy as jnp
from jax import lax
from jax.experimental import pallas as pl
from jax.experimental.pallas import tpu as pltpu
from jax.experimental.pallas import tpu_sc as plsc

NC = 2
NS = 16
NW = NC * NS

G = 128
NBUF = 2


@functools.lru_cache(maxsize=None)
def _sc_gather(n_tok, note_v, note_d, phon_d, ng, dtype):
    mesh = plsc.VectorSubcoreMesh(core_axis_name="c", subcore_axis_name="s")
    per_w = ng * G

    @functools.partial(
        pl.kernel,
        out_type=[
            jax.ShapeDtypeStruct((n_tok, note_d), dtype),
            jax.ShapeDtypeStruct((n_tok, phon_d), dtype),
        ],
        mesh=mesh,
        scratch_types=[
            pltpu.VMEM((ng, G), jnp.int32),
            pltpu.VMEM((ng, G), jnp.int32),
            pltpu.VMEM((NBUF, G, note_d), dtype),
            pltpu.VMEM((NBUF, G, phon_d), dtype),
            pltpu.SemaphoreType.DMA((NBUF,)),
        ],
    )
    def gather_kernel(idx_n_hbm, idx_p_hbm, note_hbm, phon_hbm,
                      out_n, out_p, idxn, idxp, rn, rp, sems):
        wid = lax.axis_index("s") * NC + lax.axis_index("c")
        base = wid * per_w

        pltpu.sync_copy(idx_n_hbm.at[wid], idxn)
        pltpu.sync_copy(idx_p_hbm.at[wid], idxp)

        def start(g, b):
            pltpu.async_copy(note_hbm.at[idxn.at[g]], rn.at[b], sems.at[b])
            pltpu.async_copy(phon_hbm.at[idxp.at[g]], rp.at[b], sems.at[b])

        def drain(b):
            pltpu.make_async_copy(out_n.at[pl.ds(0, G)], rn.at[b], sems.at[b]).wait()
            pltpu.make_async_copy(out_p.at[pl.ds(0, G)], rp.at[b], sems.at[b]).wait()

        def flush(g, b):
            tok = pl.multiple_of(base + g * G, G)
            pltpu.sync_copy(rn.at[b], out_n.at[pl.ds(tok, G)])
            pltpu.sync_copy(rp.at[b], out_p.at[pl.ds(tok, G)])

        for b in range(NBUF):
            start(b, b)

        def body(i, _):
            g0 = i * NBUF
            for b in range(NBUF):
                g = g0 + b
                drain(b)
                flush(g, b)

                @pl.when(g + NBUF < ng)
                def _():
                    start(g + NBUF, b)
            return 0

        lax.fori_loop(0, ng // NBUF, body, 0)

    return gather_kernel


@functools.lru_cache(maxsize=None)
def _tc_mlp(n_tok, note_d, phon_d, slur_v, slur_d, pp_v, pp_d, cond, blk,
            chunk_blks, base_blk, out_tok):
    grid = (chunk_blks,)
    nsub = blk // 128
    combo = slur_v * pp_v

    bf16 = jnp.bfloat16

    def mlp_body(*refs):
        if base_blk > 0:
            refs = refs[1:]
        n, p, c_id, s_tab, pp_tab, w1n, w1p, w1s, w1pp, b1, w2, b2, out = refs
        h = jnp.dot(n[...].astype(bf16), w1n[...].astype(bf16),
                    preferred_element_type=jnp.float32)
        h = h + jnp.dot(p[...].astype(bf16), w1p[...].astype(bf16),
                        preferred_element_type=jnp.float32)
        sw = jnp.dot(s_tab[...], w1s[...], preferred_element_type=jnp.float32)
        pw = jnp.dot(pp_tab[...], w1pp[...], preferred_element_type=jnp.float32)
        spw = (jnp.repeat(sw, pp_v, axis=0) + jnp.tile(pw, (slur_v, 1))
               ).astype(bf16)
        cid = c_id[0]
        iota_c = lax.broadcasted_iota(jnp.int32, (combo, 128), 0)
        parts = []
        for r in range(nsub):
            ohT = (cid[r:r + 1, :] == iota_c).astype(bf16)
            parts.append(lax.dot_general(
                ohT, spw, (((0,), (0,)), ((), ())),
                preferred_element_type=jnp.float32))
        h = h + jnp.concatenate(parts, axis=0)
        h = h + b1[...]
        h = h * jax.nn.sigmoid(h)
        out[...] = jnp.dot(h.astype(bf16), w2[...].astype(bf16),
                           preferred_element_type=jnp.float32) + b2[...]

    def row_spec(d):
        return pl.BlockSpec((blk, d), lambda i: (i, 0))

    def full_spec(r, c):
        return pl.BlockSpec((r, c), lambda i: (0, 0))

    in_specs = [
        row_spec(note_d), row_spec(phon_d),
        pl.BlockSpec((1, nsub, 128), lambda i: (i, 0, 0)),
        full_spec(slur_v, slur_d), full_spec(pp_v, pp_d),
        full_spec(note_d, cond), full_spec(phon_d, cond),
        full_spec(slur_d, cond), full_spec(pp_d, cond),
        full_spec(1, cond), full_spec(cond, cond), full_spec(1, cond),
    ]
    aliases = {}
    if base_blk > 0:
        in_specs = [pl.BlockSpec(memory_space=pl.MemorySpace.ANY)] + in_specs
        aliases = {0: 0}
    return pl.pallas_call(
        mlp_body,
        grid=grid,
        in_specs=in_specs,
        out_specs=pl.BlockSpec((blk, cond), lambda i: (base_blk + i, 0)),
        out_shape=jax.ShapeDtypeStruct((out_tok, cond), jnp.float32),
        input_output_aliases=aliases,
    )


def kernel(note_id, phoneme_id, slur, phone_progress, note_table,
           phoneme_table, slur_table, pp_table, W1, b1, W2, b2):
    B, L = note_id.shape
    n_tok = B * L
    note_d = note_table.shape[1]
    phon_d = phoneme_table.shape[1]
    slur_v, slur_d = slur_table.shape
    pp_v, pp_d = pp_table.shape
    cond = W2.shape[1]
    blk = 4096

    quantum = NW * G * NBUF
    assert quantum % blk == 0 or blk % quantum == 0
    if n_tok == 25 * quantum:
        sizes = [1, 2, 4, 6, 6, 6]
    else:
        sizes = [max(1, n_tok // quantum)]
    chunk_tok = [s * quantum for s in sizes]
    assert sum(chunk_tok) == n_tok
    nchunks = len(chunk_tok)
    bounds = [0]
    for t in chunk_tok:
        bounds.append(bounds[-1] + t)

    note_i32 = note_id.astype(jnp.int32).reshape(n_tok)
    phon_i32 = phoneme_id.astype(jnp.int32).reshape(n_tok)

    w1n = W1[:note_d]
    w1p = W1[note_d:note_d + phon_d]
    w1s = W1[note_d + phon_d:note_d + phon_d + slur_d]
    w1pp = W1[note_d + phon_d + slur_d:]
    b1r = b1.reshape(1, cond)
    b2r = b2.reshape(1, cond)

    c_id = (jnp.clip(slur, 0, slur_v - 1).astype(jnp.int32) * pp_v
            + phone_progress.astype(jnp.int32)).reshape(n_tok // 128, 128)

    rows = []
    for k in range(nchunks):
        ctok = chunk_tok[k]
        ng = ctok // (NW * G)
        sc = _sc_gather(ctok, note_table.shape[0], note_d, phon_d, ng,
                        jnp.float32)
        lo = bounds[k]
        idx_n = lax.dynamic_slice_in_dim(note_i32, lo, ctok).reshape(NW, ng, G)
        idx_p = lax.dynamic_slice_in_dim(phon_i32, lo, ctok).reshape(NW, ng, G)
        rows.append(sc(idx_n, idx_p, note_table, phoneme_table))

    out = None
    for k in range(nchunks):
        ctok = chunk_tok[k]
        cblks = ctok // blk
        mlp = _tc_mlp(ctok, note_d, phon_d, slur_v, slur_d, pp_v, pp_d,
                      cond, blk, cblks, bounds[k] // blk, n_tok)
        cid_k = lax.dynamic_slice_in_dim(
            c_id, bounds[k] // 128, ctok // 128).reshape(cblks, blk // 128, 128)
        args = (rows[k][0], rows[k][1], cid_k, slur_table, pp_table,
                w1n, w1p, w1s, w1pp, b1r, W2, b2r)
        out = mlp(*args) if k == 0 else mlp(out, *args)

    return out.reshape(B, L, cond)
